# baseline (speedup 1.0000x reference)
import jax
import jax.numpy as jnp
from jax.experimental import pallas as pl
from jax.experimental.pallas import tpu as pltpu

_B = 128
_L = 2048
_BIN = 128.0


def _loss_kernel(t_smem, lat_ref, tcol_ref, out_ref, y_ref):
    i = pl.program_id(0)

    @pl.when(i == 0)
    def _init():
        k = jax.lax.broadcasted_iota(jnp.int32, (1, _L), 1).astype(jnp.float32)
        norm = jnp.exp2((k + 1.0) / _BIN) - jnp.exp2(k / _BIN)
        norm_last = jnp.exp2(jnp.float32(_L) / _BIN) - jnp.exp2(
            (jnp.float32(_L) - 1.0) / _BIN)
        w = norm_last / norm
        y_ref[...] = lat_ref[...] * jnp.sqrt(w)
        out_ref[0, 0] = 0.0

    ti = t_smem[i].astype(jnp.float32)
    tom = jnp.abs(tcol_ref[...] - ti) + 1.0
    bins = jnp.ceil(jnp.log2(tom) * _BIN)
    bins = jnp.clip(bins, 0.0, float(_L)).astype(jnp.int32)

    kk = jax.lax.broadcasted_iota(jnp.int32, (_B, _L), 1)
    mask = kk >= bins

    yi = y_ref[pl.ds(i, 1), :]
    d = y_ref[...] - yi
    s = jnp.sum(jnp.where(mask, d * d, 0.0))
    out_ref[0, 0] += s

    @pl.when(i == _B - 1)
    def _fin():
        out_ref[0, 0] = out_ref[0, 0] / jnp.float32(_B * _B)


def kernel(latents, time_steps):
    t_i32 = time_steps.astype(jnp.int32)
    t_col = time_steps.astype(jnp.float32).reshape(_B, 1)
    out = pl.pallas_call(
        _loss_kernel,
        grid=(_B,),
        in_specs=[
            pl.BlockSpec(memory_space=pltpu.SMEM),
            pl.BlockSpec((_B, _L), lambda i: (0, 0)),
            pl.BlockSpec((_B, 1), lambda i: (0, 0)),
        ],
        out_specs=pl.BlockSpec(memory_space=pltpu.SMEM),
        out_shape=jax.ShapeDtypeStruct((1, 1), jnp.float32),
        scratch_shapes=[pltpu.VMEM((_B, _L), jnp.float32)],
    )(t_i32, latents, t_col)
    return out[0, 0]
